# Initial kernel scaffold; baseline (speedup 1.0000x reference)
import functools

import jax
import jax.numpy as jnp
from jax import lax
from jax.experimental import pallas as pl
from jax.experimental.pallas import tpu as pltpu
from jax.experimental.pallas import tpu_sc as plsc

NUM_NODES = 100000
NUM_SEGMENTS = 512
D = 128

NC = 2
NS = 16
NW = NC * NS
L = 16

G = 16
NGROUPS = NUM_NODES // G
GQ, GR = divmod(NGROUPS, NW)
MAX_GROUPS = GQ + 1
IDX_MAIN = GQ * G
SEG_PER_TILE = NUM_SEGMENTS // NS


def _sc_partial(v, batch):
  mesh = plsc.VectorSubcoreMesh(core_axis_name="c", subcore_axis_name="s")

  @functools.partial(
      pl.kernel,
      out_type=[
          jax.ShapeDtypeStruct((NC, NUM_SEGMENTS, D), jnp.float32),
          jax.ShapeDtypeStruct((NC, NUM_SEGMENTS, L), jnp.float32),
      ],
      mesh=mesh,
      scratch_types=[
          pltpu.VMEM((MAX_GROUPS * G,), jnp.int32),
          pltpu.VMEM((G, D), jnp.float32),
          pltpu.VMEM((G, L), jnp.float32),
          pltpu.VMEM((SEG_PER_TILE, D), jnp.float32),
          pltpu.VMEM((SEG_PER_TILE, L), jnp.float32),
          pltpu.VMEM_SHARED((NUM_SEGMENTS, D), jnp.float32),
          pltpu.VMEM_SHARED((NUM_SEGMENTS, L), jnp.float32),
      ],
  )
  def k(v_hbm, b_hbm, psum_hbm, pcnt_hbm, idx_v, rows_v, ones_v, zs_v, zc_v,
        acc_s, acc_c):
    cid = lax.axis_index("c")
    sid = lax.axis_index("s")
    wid = sid * NC + cid

    base_group = wid * GQ + jnp.minimum(wid, GR)
    ngroups = GQ + jnp.where(wid < GR, 1, 0)
    row_base = base_group * G

    zeros = jnp.zeros((L,), jnp.float32)
    ones = jnp.ones((L,), jnp.float32)

    def fill(i, _):
      ones_v[i] = ones
      zc_v[i, :] = zeros
      return 0

    lax.fori_loop(0, G, fill, 0)

    def fill2(i, _):
      zc_v[G + i, :] = zeros
      for j in range(D // L):
        zs_v[i, pl.ds(j * L, L)] = zeros
      return 0

    lax.fori_loop(0, SEG_PER_TILE, fill2, 0)

    pltpu.sync_copy(zs_v, acc_s.at[pl.ds(sid * SEG_PER_TILE, SEG_PER_TILE)])
    pltpu.sync_copy(zc_v.at[pl.ds(0, SEG_PER_TILE)],
                    acc_c.at[pl.ds(sid * SEG_PER_TILE, SEG_PER_TILE)])

    pltpu.sync_copy(b_hbm.at[pl.ds(row_base, IDX_MAIN)],
                    idx_v.at[pl.ds(0, IDX_MAIN)])

    @pl.when(wid < GR)
    def _():
      pltpu.sync_copy(b_hbm.at[pl.ds(row_base + IDX_MAIN, G)],
                      idx_v.at[pl.ds(IDX_MAIN, G)])

    plsc.subcore_barrier()

    def body(g, _):
      @pl.when(g < ngroups)
      def _():
        idx_vec = idx_v[pl.ds(g * G, L)]
        pltpu.sync_copy(v_hbm.at[pl.ds(row_base + g * G, G)], rows_v)
        pltpu.sync_copy(rows_v, acc_s.at[idx_vec], add=True)
        pltpu.sync_copy(ones_v, acc_c.at[idx_vec], add=True)
      return 0

    lax.fori_loop(0, MAX_GROUPS, body, 0)

    plsc.subcore_barrier()

    s0 = sid * SEG_PER_TILE
    pltpu.sync_copy(acc_s.at[pl.ds(s0, SEG_PER_TILE)],
                    psum_hbm.at[cid, pl.ds(s0, SEG_PER_TILE)])
    pltpu.sync_copy(acc_c.at[pl.ds(s0, SEG_PER_TILE)],
                    pcnt_hbm.at[cid, pl.ds(s0, SEG_PER_TILE)])

  return k(v, batch)


def _combine_body(u_ref, ps_ref, pc_ref, o_ref):
  s = ps_ref[0] + ps_ref[1]
  c = pc_ref[0] + pc_ref[1]
  cnt = jnp.maximum(c[:, 0:1], 1.0)
  o_ref[...] = u_ref[...] + s / cnt


def kernel(u, v, batch):
  batch = batch.astype(jnp.int32)
  psum, pcnt = _sc_partial(v, batch)
  return pl.pallas_call(
      _combine_body,
      out_shape=jax.ShapeDtypeStruct((NUM_SEGMENTS, D), jnp.float32),
  )(u, psum, pcnt)


# speedup vs baseline: 5.4113x; 5.4113x over previous
import functools

import jax
import jax.numpy as jnp
from jax import lax
from jax.experimental import pallas as pl
from jax.experimental.pallas import tpu as pltpu
from jax.experimental.pallas import tpu_sc as plsc

NUM_NODES = 100000
NUM_SEGMENTS = 512
D = 128

NC = 2
NS = 16
NW = NC * NS
L = 16
DJ = D // L

G = 16
NGROUPS = NUM_NODES // G
GQ, GR = divmod(NGROUPS, NW)
MAX_GROUPS = GQ + 1
IDX_MAIN = GQ * G
IDX_PAD = MAX_GROUPS * G

R = 3
CH = 64
CH_G = CH // G
NCH = (MAX_GROUPS + CH_G - 1) // CH_G
SUMW = NUM_SEGMENTS * D
CNTW = NUM_SEGMENTS * L


def _sc_partial(v_flat, batch):
  mesh = plsc.VectorSubcoreMesh(core_axis_name="c", subcore_axis_name="s")

  @functools.partial(
      pl.kernel,
      out_type=[
          jax.ShapeDtypeStruct((NW, SUMW), jnp.float32),
          jax.ShapeDtypeStruct((NW, CNTW), jnp.float32),
      ],
      mesh=mesh,
      scratch_types=[
          pltpu.VMEM((IDX_PAD,), jnp.int32),
          pltpu.VMEM((R * CH * D,), jnp.float32),
          pltpu.VMEM((SUMW,), jnp.float32),
          pltpu.VMEM((CNTW,), jnp.float32),
          pltpu.SemaphoreType.DMA((R,)),
      ],
  )
  def k(v_hbm, b_hbm, psum_hbm, pcnt_hbm, idx_v, ring, loc_s, loc_c, sems):
    cid = lax.axis_index("c")
    sid = lax.axis_index("s")
    wid = sid * NC + cid

    base_group = wid * GQ + jnp.minimum(wid, GR)
    ngroups = GQ + jnp.where(wid < GR, 1, 0)
    row_base = base_group * G
    nrows = ngroups * G

    zeros = jnp.zeros((L,), jnp.float32)

    def fillz(i, _):
      for j in range(DJ):
        loc_s[pl.ds(i * D + j * L, L)] = zeros
      loc_c[pl.ds(i * L, L)] = zeros
      return 0

    lax.fori_loop(0, NUM_SEGMENTS, fillz, 0)

    pltpu.sync_copy(b_hbm.at[pl.ds(row_base, IDX_MAIN)],
                    idx_v.at[pl.ds(0, IDX_MAIN)])

    @pl.when(wid < GR)
    def _():
      pltpu.sync_copy(b_hbm.at[pl.ds(row_base + IDX_MAIN, G)],
                      idx_v.at[pl.ds(IDX_MAIN, G)])

    def issue(c, slot):
      full = (c + 1) * CH <= nrows
      part = nrows - c * CH == CH - G

      @pl.when(full)
      def _():
        pltpu.async_copy(
            v_hbm.at[pl.ds((row_base + c * CH) * D, CH * D)],
            ring.at[pl.ds(slot * CH * D, CH * D)], sems.at[slot])

      @pl.when(part)
      def _():
        pltpu.async_copy(
            v_hbm.at[pl.ds((row_base + c * CH) * D, (CH - G) * D)],
            ring.at[pl.ds(slot * CH * D, (CH - G) * D)], sems.at[slot])

    def wait_chunk(c, slot):
      full = (c + 1) * CH <= nrows
      part = nrows - c * CH == CH - G

      @pl.when(full)
      def _():
        pltpu.make_async_copy(
            v_hbm.at[pl.ds((row_base + c * CH) * D, CH * D)],
            ring.at[pl.ds(slot * CH * D, CH * D)], sems.at[slot]).wait()

      @pl.when(part)
      def _():
        pltpu.make_async_copy(
            v_hbm.at[pl.ds((row_base + c * CH) * D, (CH - G) * D)],
            ring.at[pl.ds(slot * CH * D, (CH - G) * D)], sems.at[slot]).wait()

    def flush(sums, cnt, prev_s):
      for j in range(DJ):
        loc_s[pl.ds(prev_s * D + j * L, L)] = sums[j]
      loc_c[pl.ds(prev_s * L, L)] = jnp.full((L,), cnt)

    for p in range(R - 1):
      issue(jnp.int32(p), p)

    prev0 = idx_v[pl.ds(0, L)][0]
    init = (zeros, zeros, zeros, zeros, zeros, zeros, zeros, zeros,
            prev0, jnp.float32(0.0))

    def chunk_body(c, carry):
      sums = list(carry[:DJ])
      prev_s, cnt = carry[DJ], carry[DJ + 1]
      slot = lax.rem(c, R)
      wait_chunk(c, slot)
      for b in range(CH_G):
        valid = c * CH_G + b < ngroups
        vinc = jnp.where(valid, 1.0, 0.0)
        idxv = idx_v[pl.ds(c * CH + b * G, G)]
        for r in range(G):
          s = jnp.where(valid, idxv[r], prev_s)
          bnd = s != prev_s

          @pl.when(bnd)
          def _(sums=sums, prev_s=prev_s, cnt=cnt):
            flush(sums, cnt, prev_s)

          base = slot * CH * D + (b * G + r) * D
          row = [
              jnp.where(valid, ring[pl.ds(base + j * L, L)], zeros)
              for j in range(DJ)
          ]
          sums = [jnp.where(bnd, row[j], sums[j] + row[j]) for j in range(DJ)]
          cnt = jnp.where(bnd, vinc, cnt + vinc)
          prev_s = s
      issue(c + (R - 1), lax.rem(c + (R - 1), R))
      return tuple(sums) + (prev_s, cnt)

    carry = lax.fori_loop(0, NCH, chunk_body, init)

    flush(carry[:DJ], carry[DJ + 1], carry[DJ])

    pltpu.sync_copy(loc_s, psum_hbm.at[wid])
    pltpu.sync_copy(loc_c, pcnt_hbm.at[wid])

  return k(v_flat, batch)


def _combine_body(u_ref, ps_ref, pc_ref, o_ref):
  s = jnp.sum(ps_ref[...], axis=0)
  c = jnp.sum(pc_ref[...], axis=0)
  cnt = jnp.maximum(c[:, 0:1], 1.0)
  o_ref[...] = u_ref[...] + s / cnt


SEG_BLK = 128


def kernel(u, v, batch):
  batch = batch.astype(jnp.int32)
  psum, pcnt = _sc_partial(v.reshape(-1), batch)
  psum = psum.reshape(NW, NUM_SEGMENTS, D)
  pcnt = pcnt.reshape(NW, NUM_SEGMENTS, L)
  nblk = NUM_SEGMENTS // SEG_BLK
  return pl.pallas_call(
      _combine_body,
      grid=(nblk,),
      in_specs=[
          pl.BlockSpec((SEG_BLK, D), lambda i: (i, 0)),
          pl.BlockSpec((NW, SEG_BLK, D), lambda i: (0, i, 0)),
          pl.BlockSpec((NW, SEG_BLK, L), lambda i: (0, i, 0)),
      ],
      out_specs=pl.BlockSpec((SEG_BLK, D), lambda i: (i, 0)),
      out_shape=jax.ShapeDtypeStruct((NUM_SEGMENTS, D), jnp.float32),
  )(u, psum, pcnt)
